# Initial kernel scaffold; baseline (speedup 1.0000x reference)
import jax
import jax.numpy as jnp
from jax.experimental import pallas as pl


def kernel(input, target):
    raise NotImplementedError("write your pallas kernel here")


# speedup vs baseline: 3.9286x; 3.9286x over previous
import functools

import jax
import jax.numpy as jnp
from jax.experimental import pallas as pl
from jax.experimental.pallas import tpu as pltpu

_B, _C, _H, _W = 16, 6, 512, 512
_ROWS = 128
_NJ = _H // _ROWS


def _body(inp_ref, tgt_ref, wsum_ref, bits_ref):
    k = pl.program_id(0)
    j = pl.program_id(1)
    inp = inp_ref[0]
    tgt = tgt_ref[0]

    s = inp[0]
    q = inp[0] * inp[0]
    for c in range(1, _C):
        s = s + inp[c]
        q = q + inp[c] * inp[c]
    w = q - s * s * (1.0 / _C)

    m = tgt[0]
    bit = jnp.full(m.shape, 1, jnp.int32)
    for c in range(1, _C):
        gt = tgt[c] > m
        m = jnp.where(gt, tgt[c], m)
        bit = jnp.where(gt, jnp.int32(1 << c), bit)

    wsum_part = jnp.sum(jnp.where(bit > 1, w, 0.0))
    bits_part = jnp.int32(0)
    for c in range(1, _C):
        present = jnp.any(bit == (1 << c))
        bits_part = bits_part + jnp.where(present, jnp.int32(1 << c), 0)

    @pl.when(j == 0)
    def _init():
        wsum_ref[k, 0] = wsum_part
        bits_ref[k, 0] = bits_part

    @pl.when(j > 0)
    def _acc():
        wsum_ref[k, 0] = wsum_ref[k, 0] + wsum_part
        bits_ref[k, 0] = bits_ref[k, 0] | bits_part


@functools.partial(jax.jit, static_argnames=("interpret",))
def kernel(input, target, interpret=False):
    wsum, bits = pl.pallas_call(
        _body,
        grid=(_B, _NJ),
        in_specs=[
            pl.BlockSpec((1, _C, _ROWS, _W), lambda k, j: (k, 0, j, 0)),
            pl.BlockSpec((1, _C, _ROWS, _W), lambda k, j: (k, 0, j, 0)),
        ],
        out_specs=[
            pl.BlockSpec((_B, 1), lambda k, j: (0, 0), memory_space=pltpu.SMEM),
            pl.BlockSpec((_B, 1), lambda k, j: (0, 0), memory_space=pltpu.SMEM),
        ],
        out_shape=[
            jax.ShapeDtypeStruct((_B, 1), jnp.float32),
            jax.ShapeDtypeStruct((_B, 1), jnp.int32),
        ],
        interpret=interpret,
    )(input, target)
    sum_var = wsum[:, 0] * (1.0 / (_C - 1))
    n_uniq = jax.lax.population_count(bits[:, 0]).astype(jnp.float32)
    return jnp.mean(sum_var / (n_uniq + 1e-8))
